# baseline (speedup 1.0000x reference)
import jax
import jax.numpy as jnp
from jax import lax
from jax.experimental import pallas as pl
from jax.experimental.pallas import tpu as pltpu
from jax.experimental.pallas import tpu_sc as plsc

B = 16384
L = 50
V = 1000000
D_SPARSE = 3
D_DENSE = 2
D_OUT = 2

NC = 2
NS = 16
NW = NC * NS
BPW = B // NW
PASSES = 4
BPP = BPW // PASSES
IPP = BPP * L
WPP = IPP * D_SPARSE
VPT = 62496
VCH = VPT // 4
VTAIL = V - VPT * NS

_MESH = plsc.VectorSubcoreMesh(core_axis_name="c", subcore_axis_name="s")
_CP = pltpu.CompilerParams(needs_layout_passes=False, use_tc_tiling_on_sc=False)


def _sc_kernel(sp_hbm, dn_hbm, wsplat_hbm, emt_hbm, out_hbm, flat_hbm,
               sp_v, dn_v, wsplat_v, idx3_a, idx3_b, words_a, words_b,
               acc_v, out_v, col_v, tail_v, sem_a, sem_b):
    c = lax.axis_index("c")
    s = lax.axis_index("s")
    w = s * NC + c
    lanes = lax.iota(jnp.int32, 16)

    for d in range(D_SPARSE):
        for ch in range(VPT // VCH):
            off = s * VPT + ch * VCH
            pltpu.sync_copy(emt_hbm.at[d, pl.ds(off, VCH)], col_v)
            pltpu.sync_copy(col_v, flat_hbm.at[c, pl.ds(d * V + off, VCH)])

    @pl.when(s == NS - 1)
    def _():
        for d in range(D_SPARSE):
            pltpu.sync_copy(emt_hbm.at[d, pl.ds(NS * VPT, VTAIL)], tail_v)
            pltpu.sync_copy(tail_v, flat_hbm.at[c, pl.ds(d * V + NS * VPT, VTAIL)])

    pltpu.sync_copy(sp_hbm.at[pl.ds(w * BPW, BPW), :], sp_v)
    pltpu.sync_copy(dn_hbm.at[pl.ds(w * BPW, BPW), :], dn_v)
    pltpu.sync_copy(wsplat_hbm, wsplat_v)

    plsc.subcore_barrier()

    idx3 = [idx3_a, idx3_b]
    words = [words_a, words_b]
    sems = [sem_a, sem_b]

    def build(p):
        buf = idx3[p % 2]
        def body(k, _):
            q16 = k * 16 + lanes
            q = q16 + p * IPP
            vidx = plsc.load_gather(sp_v, [q // L, q % L])
            j3 = q16 * D_SPARSE
            plsc.store_scatter(buf, [j3], vidx)
            plsc.store_scatter(buf, [j3 + 1], vidx + V)
            plsc.store_scatter(buf, [j3 + 2], vidx + 2 * V)
            return 0
        lax.fori_loop(0, IPP // 16, body, 0)

    def fire(p):
        return pltpu.async_copy(
            flat_hbm.at[c].at[idx3[p % 2]], words[p % 2], sems[p % 2])

    def reduce(p):
        buf = words[p % 2]
        for g in range(BPP // 16):
            base = (g * 16 + lanes) * (L * D_SPARSE)
            for d in range(D_SPARSE):
                def rbody(l, acc):
                    return acc + plsc.load_gather(
                        buf, [base + (l * D_SPARSE + d)])
                acc = lax.fori_loop(
                    1, L, rbody, plsc.load_gather(buf, [base + d]))
                plsc.store_scatter(
                    acc_v, [(p * BPP + g * 16 + lanes) * D_SPARSE + d], acc)

    build(0)
    handles = [fire(0), None]
    for p in range(1, PASSES):
        build(p)
        handles[p % 2] = fire(p)
        handles[(p - 1) % 2].wait()
        reduce(p - 1)
    handles[(PASSES - 1) % 2].wait()
    reduce(PASSES - 1)

    wv = [wsplat_v[i, :] for i in range(2 * (D_DENSE + D_SPARSE) + D_OUT)]
    zero16 = jnp.zeros((16,), jnp.int32)
    for g in range(BPW // 16):
        r = lanes + g * 16
        d0 = plsc.load_gather(dn_v, [r, zero16])
        d1 = plsc.load_gather(dn_v, [r, zero16 + 1])
        m0 = plsc.load_gather(acc_v, [r * D_SPARSE])
        m1 = plsc.load_gather(acc_v, [r * D_SPARSE + 1])
        m2 = plsc.load_gather(acc_v, [r * D_SPARSE + 2])
        o0 = d0 * wv[0] + d1 * wv[1] + m0 * wv[2] + m1 * wv[3] + m2 * wv[4] + wv[10]
        o1 = d0 * wv[5] + d1 * wv[6] + m0 * wv[7] + m1 * wv[8] + m2 * wv[9] + wv[11]
        plsc.store_scatter(out_v, [r * D_OUT], o0)
        plsc.store_scatter(out_v, [r * D_OUT + 1], o1)

    pltpu.sync_copy(out_v, out_hbm.at[w])


@jax.jit
def kernel(dense_features, sparse_features, em_weight, W, b):
    sp = sparse_features.astype(jnp.int32)
    scale = jnp.array([1.0, 1.0, 1.0 / L, 1.0 / L, 1.0 / L], jnp.float32)
    wf = (W * scale[None, :]).reshape(-1)
    wsplat = jnp.tile(jnp.concatenate([wf, b])[:, None], (1, 16))

    run = pl.kernel(
        _sc_kernel,
        mesh=_MESH,
        compiler_params=_CP,
        out_type=(jax.ShapeDtypeStruct((NW, BPW * D_OUT), jnp.float32),
                  jax.ShapeDtypeStruct((NC, D_SPARSE * V), jnp.float32)),
        scratch_types=[
            pltpu.VMEM((BPW, L), jnp.int32),
            pltpu.VMEM((BPW, D_DENSE), jnp.float32),
            pltpu.VMEM((12, 16), jnp.float32),
            pltpu.VMEM((WPP,), jnp.int32),
            pltpu.VMEM((WPP,), jnp.int32),
            pltpu.VMEM((WPP,), jnp.float32),
            pltpu.VMEM((WPP,), jnp.float32),
            pltpu.VMEM((BPW * D_SPARSE,), jnp.float32),
            pltpu.VMEM((BPW * D_OUT,), jnp.float32),
            pltpu.VMEM((VCH,), jnp.float32),
            pltpu.VMEM((VTAIL,), jnp.float32),
            pltpu.SemaphoreType.DMA,
            pltpu.SemaphoreType.DMA,
        ],
    )
    out, _ = run(sp, dense_features, wsplat, em_weight.T)
    return out.reshape(B, D_OUT)


# speedup vs baseline: 1.1382x; 1.1382x over previous
import jax
import jax.numpy as jnp
from jax import lax
from jax.experimental import pallas as pl
from jax.experimental.pallas import tpu as pltpu
from jax.experimental.pallas import tpu_sc as plsc

B = 16384
L = 50
V = 1000000
D_SPARSE = 3
D_DENSE = 2
D_OUT = 2

NC = 2
NS = 16
NW = NC * NS
BPW = B // NW
PASSES = 4
BPP = BPW // PASSES
IPP = BPP * L
WPP = IPP * D_SPARSE
VPT = 62496
VCH = VPT // 4
VTAIL = V - VPT * NS

_MESH = plsc.VectorSubcoreMesh(core_axis_name="c", subcore_axis_name="s")
_CP = pltpu.CompilerParams(needs_layout_passes=False, use_tc_tiling_on_sc=False)


def _sc_kernel(sp_hbm, dn_hbm, wsplat_hbm, emt_hbm, out_hbm, flat_hbm,
               sp_v, dn_v, wsplat_v, idx3_a, idx3_b, words_a, words_b,
               acc_v, out_v, col_v, tail_v, sem_a, sem_b):
    c = lax.axis_index("c")
    s = lax.axis_index("s")
    w = s * NC + c
    lanes = lax.iota(jnp.int32, 16)

    with jax.named_scope("tbl_copy"):
        for d in range(D_SPARSE):
            for ch in range(VPT // VCH):
                off = s * VPT + ch * VCH
                pltpu.sync_copy(emt_hbm.at[d, pl.ds(off, VCH)], col_v)
                pltpu.sync_copy(col_v, flat_hbm.at[c, pl.ds(d * V + off, VCH)])

    @pl.when(s == NS - 1)
    def _():
        for d in range(D_SPARSE):
            pltpu.sync_copy(emt_hbm.at[d, pl.ds(NS * VPT, VTAIL)], tail_v)
            pltpu.sync_copy(tail_v, flat_hbm.at[c, pl.ds(d * V + NS * VPT, VTAIL)])

    with jax.named_scope("stage"):
        pltpu.sync_copy(sp_hbm.at[:, pl.ds(w * BPW, BPW)], sp_v)
        pltpu.sync_copy(dn_hbm.at[:, pl.ds(w * BPW, BPW)], dn_v)
        pltpu.sync_copy(wsplat_hbm, wsplat_v)

    plsc.subcore_barrier()

    idx3 = [idx3_a, idx3_b]
    words = [words_a, words_b]
    sems = [sem_a, sem_b]

    def build(p):
        buf = idx3[p % 2]
        def body(k, _):
            q16 = k * 16 + lanes
            q = q16 + p * IPP
            vidx = plsc.load_gather(sp_v, [q % L, q // L])
            j3 = q16 * D_SPARSE
            plsc.store_scatter(buf, [j3], vidx)
            plsc.store_scatter(buf, [j3 + 1], vidx + V)
            plsc.store_scatter(buf, [j3 + 2], vidx + 2 * V)
            return 0
        with jax.named_scope("build"):
            lax.fori_loop(0, IPP // 16, body, 0)

    def fire(p):
        return pltpu.async_copy(
            flat_hbm.at[c].at[idx3[p % 2]], words[p % 2], sems[p % 2])

    def reduce(p):
        buf = words[p % 2]
        for g in range(BPP // 16):
            base = (g * 16 + lanes) * (L * D_SPARSE)
            for d in range(D_SPARSE):
                def rbody(l, acc):
                    return acc + plsc.load_gather(
                        buf, [base + (l * D_SPARSE + d)])
                acc = lax.fori_loop(
                    1, L, rbody, plsc.load_gather(buf, [base + d]))
                plsc.store_scatter(
                    acc_v, [(p * BPP + g * 16 + lanes) * D_SPARSE + d], acc)

    build(0)
    handles = [fire(0), None]
    for p in range(1, PASSES):
        build(p)
        handles[p % 2] = fire(p)
        with jax.named_scope("gwait"):
            handles[(p - 1) % 2].wait()
        with jax.named_scope("reduce"):
            reduce(p - 1)
    with jax.named_scope("gwait"):
        handles[(PASSES - 1) % 2].wait()
    with jax.named_scope("reduce"):
        reduce(PASSES - 1)

    wv = [wsplat_v[i, :] for i in range(2 * (D_DENSE + D_SPARSE) + D_OUT)]

    zero16 = jnp.zeros((16,), jnp.int32)
    for g in range(BPW // 16):
        r = lanes + g * 16
        d0 = dn_v[0, pl.ds(g * 16, 16)]
        d1 = dn_v[1, pl.ds(g * 16, 16)]
        m0 = plsc.load_gather(acc_v, [r * D_SPARSE])
        m1 = plsc.load_gather(acc_v, [r * D_SPARSE + 1])
        m2 = plsc.load_gather(acc_v, [r * D_SPARSE + 2])
        o0 = d0 * wv[0] + d1 * wv[1] + m0 * wv[2] + m1 * wv[3] + m2 * wv[4] + wv[10]
        o1 = d0 * wv[5] + d1 * wv[6] + m0 * wv[7] + m1 * wv[8] + m2 * wv[9] + wv[11]
        plsc.store_scatter(out_v, [r * D_OUT], o0)
        plsc.store_scatter(out_v, [r * D_OUT + 1], o1)

    pltpu.sync_copy(out_v, out_hbm.at[w])


@jax.jit
def kernel(dense_features, sparse_features, em_weight, W, b):
    sp = sparse_features.astype(jnp.int32)
    scale = jnp.array([1.0, 1.0, 1.0 / L, 1.0 / L, 1.0 / L], jnp.float32)
    wf = (W * scale[None, :]).reshape(-1)
    wsplat = jnp.tile(jnp.concatenate([wf, b])[:, None], (1, 16))

    run = pl.kernel(
        _sc_kernel,
        mesh=_MESH,
        compiler_params=_CP,
        out_type=(jax.ShapeDtypeStruct((NW, BPW * D_OUT), jnp.float32),
                  jax.ShapeDtypeStruct((NC, D_SPARSE * V), jnp.float32)),
        scratch_types=[
            pltpu.VMEM((L, BPW), jnp.int32),
            pltpu.VMEM((D_DENSE, BPW), jnp.float32),
            pltpu.VMEM((12, 16), jnp.float32),
            pltpu.VMEM((WPP,), jnp.int32),
            pltpu.VMEM((WPP,), jnp.int32),
            pltpu.VMEM((WPP,), jnp.float32),
            pltpu.VMEM((WPP,), jnp.float32),
            pltpu.VMEM((BPW * D_SPARSE,), jnp.float32),
            pltpu.VMEM((BPW * D_OUT,), jnp.float32),
            pltpu.VMEM((VCH,), jnp.float32),
            pltpu.VMEM((VTAIL,), jnp.float32),
            pltpu.SemaphoreType.DMA,
            pltpu.SemaphoreType.DMA,
        ],
    )
    out, _ = run(sp.T, dense_features.T, wsplat, em_weight.T)
    return out.reshape(B, D_OUT)
